# baseline (speedup 1.0000x reference)
import functools

import jax
import jax.numpy as jnp
import numpy as np
from jax import lax
from jax.experimental import pallas as pl
from jax.experimental.pallas import tpu as pltpu
from jax.experimental.pallas import tpu_sc as plsc

_SEQ = 512
_DIAG = 2
_N = (_SEQ - _DIAG) * (_SEQ - _DIAG + 1) // 2
_ROWS = 2 * 64
_FLAT = _SEQ * _SEQ

_NW = 32
_ROWS_PER_W = _ROWS // _NW
_CHUNK = 16288
_NCHUNK = _N // _CHUNK
_TAIL_OFF = _NCHUNK * _CHUNK
_IDX_PAD = _TAIL_OFF + 8


def _triu_index() -> np.ndarray:
    i, j = np.triu_indices(_SEQ, _DIAG)
    idx = (i + _SEQ * j).astype(np.int32)
    pad = np.full(_IDX_PAD - _N, idx[-1], np.int32)
    return np.concatenate([idx, pad])


_IDX = _triu_index()


def _sc_gather_body(x_hbm, idx_hbm, out_hbm, idx_v, buf_v, idx8_v, buf8_v, sem):
    wid = lax.axis_index("s") * 2 + lax.axis_index("c")
    row0 = wid * _ROWS_PER_W

    def chunk_loop(c, carry):
        pltpu.sync_copy(idx_hbm.at[pl.ds(c * _CHUNK, _CHUNK)], idx_v)

        def row_loop(k, carry2):
            r = row0 + k
            row = x_hbm.at[r]
            pltpu.async_copy(row.at[idx_v], buf_v, sem).wait()
            pltpu.sync_copy(buf_v, out_hbm.at[r, pl.ds(c * _CHUNK, _CHUNK)])
            return carry2

        return lax.fori_loop(0, _ROWS_PER_W, row_loop, carry)

    lax.fori_loop(0, _NCHUNK, chunk_loop, 0)

    pltpu.sync_copy(idx_hbm.at[pl.ds(_TAIL_OFF, 8)], idx8_v)

    def tail_loop(k, carry):
        r = row0 + k
        row = x_hbm.at[r]
        pltpu.async_copy(row.at[idx8_v], buf8_v, sem).wait()
        pltpu.sync_copy(buf8_v.at[pl.ds(0, 1)],
                        out_hbm.at[r, pl.ds(_TAIL_OFF, 1)])
        return carry

    lax.fori_loop(0, _ROWS_PER_W, tail_loop, 0)


@jax.jit
def _sc_gather(x, idx):
    mesh = plsc.VectorSubcoreMesh(core_axis_name="c", subcore_axis_name="s")
    f = functools.partial(
        pl.kernel,
        out_type=jax.ShapeDtypeStruct((_ROWS, _N), jnp.float32),
        mesh=mesh,
        scratch_types=[
            pltpu.VMEM((_CHUNK,), jnp.int32),
            pltpu.VMEM((_CHUNK,), jnp.float32),
            pltpu.VMEM((8,), jnp.int32),
            pltpu.VMEM((8,), jnp.float32),
            pltpu.SemaphoreType.DMA,
        ],
        compiler_params=pltpu.CompilerParams(use_tc_tiling_on_sc=False),
    )(_sc_gather_body)
    return f(x, idx)


def kernel(inputs):
    b, c, s, _ = inputs.shape
    x = inputs.reshape(b * c, s * s)
    idx = jnp.asarray(_IDX)
    out = _sc_gather(x, idx)
    return out.reshape(b, c, _N)


# speedup vs baseline: 1.0183x; 1.0183x over previous
import functools

import jax
import jax.numpy as jnp
from jax import lax
from jax.experimental import pallas as pl
from jax.experimental.pallas import tpu as pltpu
from jax.experimental.pallas import tpu_sc as plsc

_SEQ = 512
_N = 130305
_ROWS = 128
_CB = 16
_STAGE = _CB * _SEQ
_OUTBUF = 8104
_UNROLL = 4


def _off(i):
    return (i * (1021 - i)) // 2


def _transpose_body(x_ref, o_ref):
    o_ref[...] = jnp.swapaxes(x_ref[...], 1, 2)


@jax.jit
def _tc_transpose(x):
    return pl.pallas_call(
        _transpose_body,
        grid=(_ROWS,),
        in_specs=[pl.BlockSpec((1, _SEQ, _SEQ), lambda g: (g, 0, 0))],
        out_specs=pl.BlockSpec((1, _SEQ, _SEQ), lambda g: (g, 0, 0)),
        out_shape=jax.ShapeDtypeStruct((_ROWS, _SEQ, _SEQ), jnp.float32),
    )(x)


def _pack_body(t_hbm, out_hbm, stage_v, outbuf_v):
    w = lax.axis_index("s") * 2 + lax.axis_index("c")

    for phase in range(2):
        cb = w if phase == 0 else 31 - w
        c0 = cb * _CB
        out0 = pl.multiple_of(_off(c0), 8)
        seglen = 8040 - 256 * cb + (cb // 31)

        def row_body(k, carry):
            r = phase * 64 + k
            pltpu.sync_copy(
                t_hbm.at[r, pl.ds(pl.multiple_of(c0 * _SEQ, 8), _STAGE)],
                stage_v.at[pl.ds(0, _STAGE)])

            for ii in range(_CB):
                i = c0 + ii
                length = 510 - i
                dst0 = _off(i) - out0
                src0 = 512 * ii + i + 2
                nv = lax.max((length + 16 * _UNROLL - 1) // (16 * _UNROLL),
                             0)

                def vcopy(t, carry1):
                    for u in range(_UNROLL):
                        d = t * (16 * _UNROLL) + 16 * u
                        outbuf_v[pl.ds(dst0 + d, 16)] = (
                            stage_v[pl.ds(src0 + d, 16)])
                    return carry1

                lax.fori_loop(0, nv, vcopy, 0)

            n512 = seglen // 512

            def out_chunk(t, carry2):
                pltpu.sync_copy(
                    outbuf_v.at[pl.ds(t * 512, 512)],
                    out_hbm.at[r, pl.ds(pl.multiple_of(out0 + t * 512, 8),
                                        512)])
                return carry2

            lax.fori_loop(0, n512, out_chunk, 0)

            rem = seglen - n512 * 512

            @pl.when(jnp.logical_and(rem > 0, seglen >= 512))
            def _():
                pltpu.sync_copy(
                    outbuf_v.at[pl.ds(pl.multiple_of(seglen - 512, 8), 512)],
                    out_hbm.at[r, pl.ds(pl.multiple_of(out0 + seglen - 512,
                                                       8), 512)])

            @pl.when(seglen < 512)
            def _():
                def piece(t, carry3):
                    pltpu.sync_copy(
                        outbuf_v.at[pl.ds(t * 8, 8)],
                        out_hbm.at[r, pl.ds(pl.multiple_of(out0 + t * 8, 8),
                                            8)])
                    return carry3

                lax.fori_loop(0, rem // 8, piece, 0)

                @pl.when(rem % 8 != 0)
                def _():
                    last = pl.multiple_of((rem // 8) * 8, 8)
                    pltpu.sync_copy(
                        outbuf_v.at[pl.ds(last, 1)],
                        out_hbm.at[r, pl.ds(pl.multiple_of(out0 + last, 8),
                                            1)])

            return carry

        lax.fori_loop(0, 64, row_body, 0)


@jax.jit
def _sc_pack(t):
    mesh = plsc.VectorSubcoreMesh(core_axis_name="c", subcore_axis_name="s")
    f = functools.partial(
        pl.kernel,
        out_type=jax.ShapeDtypeStruct((_ROWS, _N), jnp.float32),
        mesh=mesh,
        scratch_types=[
            pltpu.VMEM((_STAGE + 16 * _UNROLL,), jnp.float32),
            pltpu.VMEM((_OUTBUF,), jnp.float32),
        ],
        compiler_params=pltpu.CompilerParams(use_tc_tiling_on_sc=False),
    )(_pack_body)
    return f(t)


def kernel(inputs):
    b, c, s, _ = inputs.shape
    x = inputs.reshape(b * c, s, s)
    t = _tc_transpose(x).reshape(_ROWS, s * s)
    out = _sc_pack(t)
    return out.reshape(b, c, _N)


# speedup vs baseline: 1.0225x; 1.0041x over previous
import functools

import jax
import jax.numpy as jnp
from jax import lax
from jax.experimental import pallas as pl
from jax.experimental.pallas import tpu as pltpu
from jax.experimental.pallas import tpu_sc as plsc

_SEQ = 512
_N = 130305
_NPAD = 130312
_ROWS = 128
_CB = 16
_STAGE = _CB * _SEQ
_OUTBUF = 8104
_UNROLL = 4


def _off(i):
    return (i * (1021 - i)) // 2


def _transpose_body(x_ref, o_ref):
    o_ref[...] = jnp.swapaxes(x_ref[...], 1, 2)


@jax.jit
def _tc_transpose(x):
    return pl.pallas_call(
        _transpose_body,
        grid=(_ROWS,),
        in_specs=[pl.BlockSpec((1, _SEQ, _SEQ), lambda g: (g, 0, 0))],
        out_specs=pl.BlockSpec((1, _SEQ, _SEQ), lambda g: (g, 0, 0)),
        out_shape=jax.ShapeDtypeStruct((_ROWS, _SEQ, _SEQ), jnp.float32),
    )(x)


def _pack_body(t_hbm, out_hbm, stage_v, outbuf_v):
    w = lax.axis_index("s") * 2 + lax.axis_index("c")

    for phase in range(2):
        cb = w if phase == 0 else 31 - w
        c0 = cb * _CB
        out0 = pl.multiple_of(_off(c0), 8)
        seglen = 8040 - 256 * cb + 8 * (cb // 31)

        def row_body(k, carry):
            r = phase * 64 + k
            pltpu.sync_copy(
                t_hbm.at[r, pl.ds(pl.multiple_of(c0 * _SEQ, 8), _STAGE)],
                stage_v.at[pl.ds(0, _STAGE)])

            for ii in range(_CB):
                i = c0 + ii
                length = 510 - i
                dst0 = _off(i) - out0
                src0 = 512 * ii + i + 2
                nv = lax.max((length + 16 * _UNROLL - 1) // (16 * _UNROLL),
                             0)

                def vcopy(t, carry1):
                    for u in range(_UNROLL):
                        d = t * (16 * _UNROLL) + 16 * u
                        outbuf_v[pl.ds(dst0 + d, 16)] = (
                            stage_v[pl.ds(src0 + d, 16)])
                    return carry1

                lax.fori_loop(0, nv, vcopy, 0)

            n512 = seglen // 512

            def out_chunk(t, carry2):
                pltpu.sync_copy(
                    outbuf_v.at[pl.ds(t * 512, 512)],
                    out_hbm.at[r, pl.ds(pl.multiple_of(out0 + t * 512, 8),
                                        512)])
                return carry2

            lax.fori_loop(0, n512, out_chunk, 0)

            rem = seglen - n512 * 512

            @pl.when(jnp.logical_and(rem > 0, seglen >= 512))
            def _():
                pltpu.sync_copy(
                    outbuf_v.at[pl.ds(pl.multiple_of(seglen - 512, 8), 512)],
                    out_hbm.at[r, pl.ds(pl.multiple_of(out0 + seglen - 512,
                                                       8), 512)])

            @pl.when(seglen < 512)
            def _():
                def piece(t, carry3):
                    pltpu.sync_copy(
                        outbuf_v.at[pl.ds(t * 8, 8)],
                        out_hbm.at[r, pl.ds(pl.multiple_of(out0 + t * 8, 8),
                                            8)])
                    return carry3

                lax.fori_loop(0, rem // 8, piece, 0)

                @pl.when(rem % 8 != 0)
                def _():
                    last = pl.multiple_of((rem // 8) * 8, 8)
                    pltpu.sync_copy(
                        outbuf_v.at[pl.ds(last, 1)],
                        out_hbm.at[r, pl.ds(pl.multiple_of(out0 + last, 8),
                                            1)])

            return carry

        lax.fori_loop(0, 64, row_body, 0)


@jax.jit
def _sc_pack(t):
    mesh = plsc.VectorSubcoreMesh(core_axis_name="c", subcore_axis_name="s")
    f = functools.partial(
        pl.kernel,
        out_type=jax.ShapeDtypeStruct((_ROWS, _NPAD), jnp.float32),
        mesh=mesh,
        scratch_types=[
            pltpu.VMEM((_STAGE + 16 * _UNROLL,), jnp.float32),
            pltpu.VMEM((_OUTBUF,), jnp.float32),
        ],
        compiler_params=pltpu.CompilerParams(use_tc_tiling_on_sc=False),
    )(_pack_body)
    return f(t)


def kernel(inputs):
    b, c, s, _ = inputs.shape
    x = inputs.reshape(b * c, s, s)
    t = _tc_transpose(x).reshape(_ROWS, s * s)
    out = _sc_pack(t)
    return out[:, :_N].reshape(b, c, _N)
